# Initial kernel scaffold; baseline (speedup 1.0000x reference)
import jax
import jax.numpy as jnp
from jax.experimental import pallas as pl


def kernel(x, edge_index, edge_attr, batch, We1, be1, W1a, b1a, W1b, b1b, We2, be2, W2a, b2a, W2b, b2b, Wf0, bf0, Wf1, bf1, Wf2, bf2, Wr, br):
    raise NotImplementedError("write your pallas kernel here")


# speedup vs baseline: 2.9065x; 2.9065x over previous
import functools

import jax
import jax.numpy as jnp
from jax import lax
from jax.experimental import pallas as pl
from jax.experimental.pallas import tpu as pltpu
from jax.experimental.pallas import tpu_sc as plsc

N = 10000
E = 320000
D = 128
ED = 16
G = 64

NC = 2
NS = 16
NW = NC * NS
CH = 128
NCHUNK = E // CH
NP_ = 10240
ROWS_PER_TILE = NP_ // NS
RCOPY = 128


def _leaky(v):
    return jnp.where(v > 0, v, v * jnp.float32(0.01))


def _make_sc_agg(Df):
    mesh = plsc.VectorSubcoreMesh(
        core_axis_name="c", subcore_axis_name="s", num_cores=NC, num_subcores=NS
    )
    kvecs = Df // 16

    @functools.partial(
        pl.kernel,
        out_type=jax.ShapeDtypeStruct((NC * NP_, Df), jnp.float32),
        mesh=mesh,
        scratch_types=[
            pltpu.VMEM((CH,), jnp.int32),
            pltpu.VMEM((CH,), jnp.int32),
            pltpu.VMEM((CH, Df), jnp.float32),
            pltpu.VMEM((CH, Df), jnp.float32),
            pltpu.VMEM_SHARED((NP_, Df), jnp.float32),
            pltpu.SemaphoreType.DMA,
        ],
        compiler_params=pltpu.CompilerParams(use_tc_tiling_on_sc=False),
    )
    def sc_agg(x_hbm, src_hbm, dst_hbm, e_hbm, out_hbm, sidx, didx, rows, evs, agg, sem):
        c = lax.axis_index("c")
        s = lax.axis_index("s")
        wid = s * NC + c

        row0 = s * ROWS_PER_TILE

        def zero_row(i, carry):
            for k in range(kvecs):
                rows[i, pl.ds(k * 16, 16)] = jnp.zeros((16,), jnp.float32)
            return carry

        lax.fori_loop(0, RCOPY, zero_row, 0)
        for k in range(ROWS_PER_TILE // RCOPY):
            pltpu.sync_copy(rows.at[pl.ds(0, RCOPY)],
                            agg.at[pl.ds(row0 + k * RCOPY, RCOPY)])
        plsc.subcore_barrier()

        nj = NCHUNK // NW + jnp.where(wid < NCHUNK % NW, 1, 0)

        def chunk_body(j, carry):
            base = (wid + j * NW) * CH
            pltpu.sync_copy(src_hbm.at[pl.ds(base, CH)], sidx)
            pltpu.sync_copy(dst_hbm.at[pl.ds(base, CH)], didx)
            pltpu.async_copy(x_hbm.at[sidx], rows, sem).wait()
            pltpu.sync_copy(e_hbm.at[pl.ds(base, CH)], evs)

            def relu_row(r, inner):
                for k in range(kvecs):
                    sl = pl.ds(k * 16, 16)
                    rows[r, sl] = jnp.maximum(rows[r, sl] + evs[r, sl],
                                              jnp.float32(0.0))
                return inner

            lax.fori_loop(0, CH, relu_row, 0)
            pltpu.sync_copy(rows, agg.at[didx], add=True)
            return carry

        lax.fori_loop(0, nj, chunk_body, 0)
        plsc.subcore_barrier()

        for k in range(ROWS_PER_TILE // RCOPY):
            r = row0 + k * RCOPY
            pltpu.sync_copy(agg.at[pl.ds(r, RCOPY)], rows.at[pl.ds(0, RCOPY)])
            pltpu.sync_copy(rows.at[pl.ds(0, RCOPY)],
                            out_hbm.at[pl.ds(c * NP_ + r, RCOPY)])

    return sc_agg


_sc_agg_128 = _make_sc_agg(128)
_sc_agg_64 = _make_sc_agg(64)


_EB = 2000


def _edge_mlp_body(ea, We1, be1, We2, be2, e1, e2):
    a = ea[...]
    e1[...] = jnp.dot(a, We1[...], preferred_element_type=jnp.float32) + be1[...]
    e2[...] = jnp.dot(a, We2[...], preferred_element_type=jnp.float32) + be2[...]


def _edge_mlp(ea, We1, be1, We2, be2):
    grid = (E // _EB,)
    return pl.pallas_call(
        _edge_mlp_body,
        grid=grid,
        in_specs=[
            pl.BlockSpec((_EB, ED), lambda i: (i, 0)),
            pl.BlockSpec((ED, D), lambda i: (0, 0)),
            pl.BlockSpec((1, D), lambda i: (0, 0)),
            pl.BlockSpec((ED, 64), lambda i: (0, 0)),
            pl.BlockSpec((1, 64), lambda i: (0, 0)),
        ],
        out_specs=[
            pl.BlockSpec((_EB, D), lambda i: (i, 0)),
            pl.BlockSpec((_EB, 64), lambda i: (i, 0)),
        ],
        out_shape=[
            jax.ShapeDtypeStruct((E, D), jnp.float32),
            jax.ShapeDtypeStruct((E, 64), jnp.float32),
        ],
    )(ea, We1, be1, We2, be2)


_NB = 1000


def _node_mlp1_body(x, a0, a1, W1a, b1a, W1b, b1b, h1):
    t = x[...] + a0[...] + a1[...]
    u = _leaky(jnp.dot(t, W1a[...], preferred_element_type=jnp.float32) + b1a[...])
    h1[...] = _leaky(jnp.dot(u, W1b[...], preferred_element_type=jnp.float32) + b1b[...])


def _node_mlp1(x, a0, a1, W1a, b1a, W1b, b1b):
    grid = (N // _NB,)
    return pl.pallas_call(
        _node_mlp1_body,
        grid=grid,
        in_specs=[
            pl.BlockSpec((_NB, D), lambda i: (i, 0)),
            pl.BlockSpec((_NB, D), lambda i: (i, 0)),
            pl.BlockSpec((_NB, D), lambda i: (i, 0)),
            pl.BlockSpec((D, 32), lambda i: (0, 0)),
            pl.BlockSpec((1, 32), lambda i: (0, 0)),
            pl.BlockSpec((32, 64), lambda i: (0, 0)),
            pl.BlockSpec((1, 64), lambda i: (0, 0)),
        ],
        out_specs=pl.BlockSpec((_NB, 64), lambda i: (i, 0)),
        out_shape=jax.ShapeDtypeStruct((N, 64), jnp.float32),
    )(x, a0, a1, W1a, b1a, W1b, b1b)


def _node_mlp2_pool_body(h1, a0, a1, W2a, b2a, W2b, b2b, batch, g):
    i = pl.program_id(0)
    t = h1[...] + a0[...] + a1[...]
    u = _leaky(jnp.dot(t, W2a[...], preferred_element_type=jnp.float32) + b2a[...])
    h2 = _leaky(jnp.dot(u, W2b[...], preferred_element_type=jnp.float32) + b2b[...])
    onehot = (batch[...] == lax.broadcasted_iota(jnp.int32, (1, G), 1)
              ).astype(jnp.float32)
    contrib = lax.dot_general(onehot, h2, (((0,), (0,)), ((), ())),
                              preferred_element_type=jnp.float32)

    @pl.when(i == 0)
    def _():
        g[...] = contrib

    @pl.when(i > 0)
    def _():
        g[...] = g[...] + contrib


def _node_mlp2_pool(h1, a0, a1, W2a, b2a, W2b, b2b, batch2d):
    grid = (N // _NB,)
    return pl.pallas_call(
        _node_mlp2_pool_body,
        grid=grid,
        in_specs=[
            pl.BlockSpec((_NB, 64), lambda i: (i, 0)),
            pl.BlockSpec((_NB, 64), lambda i: (i, 0)),
            pl.BlockSpec((_NB, 64), lambda i: (i, 0)),
            pl.BlockSpec((64, 128), lambda i: (0, 0)),
            pl.BlockSpec((1, 128), lambda i: (0, 0)),
            pl.BlockSpec((128, 256), lambda i: (0, 0)),
            pl.BlockSpec((1, 256), lambda i: (0, 0)),
            pl.BlockSpec((_NB, 1), lambda i: (i, 0)),
        ],
        out_specs=pl.BlockSpec((G, 256), lambda i: (0, 0)),
        out_shape=jax.ShapeDtypeStruct((G, 256), jnp.float32),
    )(h1, a0, a1, W2a, b2a, W2b, b2b, batch2d)


def _head_body(g, Wf0, bf0, Wf1, bf1, Wf2, bf2, Wr, br, out):
    t = _leaky(jnp.dot(g[...], Wf0[...], preferred_element_type=jnp.float32) + bf0[...])
    t = _leaky(jnp.dot(t, Wf1[...], preferred_element_type=jnp.float32) + bf1[...])
    t = _leaky(jnp.dot(t, Wf2[...], preferred_element_type=jnp.float32) + bf2[...])
    out[...] = jnp.dot(t, Wr[...], preferred_element_type=jnp.float32) + br[...]


def _head(g, Wf0, bf0, Wf1, bf1, Wf2, bf2, Wr, br):
    return pl.pallas_call(
        _head_body,
        out_shape=jax.ShapeDtypeStruct((G, 1), jnp.float32),
    )(g, Wf0, bf0, Wf1, bf1, Wf2, bf2, Wr, br)


def kernel(x, edge_index, edge_attr, batch, We1, be1, W1a, b1a, W1b, b1b,
           We2, be2, W2a, b2a, W2b, b2b, Wf0, bf0, Wf1, bf1, Wf2, bf2, Wr, br):
    src = edge_index[0]
    dst = edge_index[1]
    batch2d = batch.reshape(N, 1)

    e1, e2 = _edge_mlp(edge_attr, We1, be1.reshape(1, D), We2, be2.reshape(1, 64))

    p1 = _sc_agg_128(x, src, dst, e1)
    h1 = _node_mlp1(x, p1[:N], p1[NP_:NP_ + N], W1a, b1a.reshape(1, 32),
                    W1b, b1b.reshape(1, 64))

    p2 = _sc_agg_64(h1, src, dst, e2)
    g = _node_mlp2_pool(h1, p2[:N], p2[NP_:NP_ + N], W2a, b2a.reshape(1, 128),
                        W2b, b2b.reshape(1, 256), batch2d)

    out = _head(g, Wf0, bf0.reshape(1, 128), Wf1, bf1.reshape(1, 64),
                Wf2, bf2.reshape(1, 32), Wr, br.reshape(1, 1))
    return out.reshape(G)
